# baseline (speedup 1.0000x reference)
import functools

import jax
import jax.numpy as jnp
from jax import lax
from jax.experimental import pallas as pl
from jax.experimental.pallas import tpu as pltpu
from jax.experimental.pallas import tpu_sc as plsc

NC = 2
NS = 16
NW = NC * NS
LANES = 16


def _mesh():
    return plsc.VectorSubcoreMesh(
        core_axis_name="c", subcore_axis_name="s",
        num_cores=NC, num_subcores=NS)


def _edge_chunk(per):
    for c in range(128, 0, -8):
        if per % c == 0:
            return c
    raise ValueError(f"per-worker edge count {per} has no 8-multiple divisor")


def _deg_kernel(E, NP2):
    per = E // NW
    C = _edge_chunk(per)
    NCH = per // C
    RPS = NP2 // NS

    assert NCH % 2 == 1

    @functools.partial(
        pl.kernel,
        out_type=jax.ShapeDtypeStruct((NC * NP2,), jnp.float32),
        mesh=_mesh(),
        scratch_types=[
            pltpu.VMEM((C,), jnp.int32),
            pltpu.VMEM((C,), jnp.int32),
            pltpu.VMEM((C,), jnp.float32),
            pltpu.VMEM((RPS,), jnp.float32),
            pltpu.VMEM_SHARED((NP2,), jnp.float32),
            pltpu.SemaphoreType.DMA,
            pltpu.SemaphoreType.DMA,
        ],
    )
    def deg(dst_hbm, out_hbm, didx0, didx1, ones, zb, acc, isem0, isem1):
        cid = lax.axis_index("c")
        sid = lax.axis_index("s")
        wid = cid * NS + sid
        wbase = wid * per
        for k in range(C // LANES):
            ones[pl.ds(k * LANES, LANES)] = jnp.full((LANES,), 1.0, jnp.float32)

        def zb_body(k, carry):
            zb[pl.ds(k * LANES, LANES)] = jnp.zeros((LANES,), jnp.float32)
            return carry
        lax.fori_loop(0, RPS // LANES, zb_body, 0)
        pltpu.sync_copy(zb, acc.at[pl.ds(sid * RPS, RPS)])
        plsc.subcore_barrier()

        pltpu.async_copy(dst_hbm.at[pl.ds(wbase, C)], didx0, isem0)

        def body(jj, carry):
            j0 = 2 * jj
            pltpu.async_copy(dst_hbm.at[pl.ds(wbase + (j0 + 1) * C, C)], didx1, isem1)
            pltpu.make_async_copy(dst_hbm.at[pl.ds(wbase, C)], didx0, isem0).wait()
            pltpu.sync_copy(ones, acc.at[didx0], add=True)
            pltpu.async_copy(dst_hbm.at[pl.ds(wbase + (j0 + 2) * C, C)], didx0, isem0)
            pltpu.make_async_copy(dst_hbm.at[pl.ds(wbase, C)], didx1, isem1).wait()
            pltpu.sync_copy(ones, acc.at[didx1], add=True)
            return carry
        lax.fori_loop(0, (NCH - 1) // 2, body, 0)

        pltpu.make_async_copy(dst_hbm.at[pl.ds(wbase, C)], didx0, isem0).wait()
        pltpu.sync_copy(ones, acc.at[didx0], add=True)
        plsc.subcore_barrier()

        pltpu.sync_copy(acc.at[pl.ds(sid * RPS, RPS)], zb)
        pltpu.sync_copy(zb, out_hbm.at[pl.ds(cid * NP2 + sid * RPS, RPS)])

    return deg


def _edge_kernel(N, D, E, NPAD):
    per = E // NW
    C = _edge_chunk(per)
    NCH = per // C
    RPS = NPAD // NS
    CP = C
    assert RPS % CP == 0

    assert NCH % 2 == 1, "pipelined edge loop expects an odd chunk count"

    @functools.partial(
        pl.kernel,
        out_type=jax.ShapeDtypeStruct((NC * NPAD, D), jnp.float32),
        mesh=_mesh(),
        scratch_types=[
            pltpu.VMEM((per,), jnp.int32),
            pltpu.VMEM((C,), jnp.int32),
            pltpu.VMEM((C,), jnp.int32),
            pltpu.VMEM((C, D), jnp.float32),
            pltpu.VMEM((C, D), jnp.float32),
            pltpu.VMEM_SHARED((NPAD, D), jnp.float32),
            pltpu.SemaphoreType.DMA,
            pltpu.SemaphoreType.DMA,
            pltpu.SemaphoreType.DMA,
            pltpu.SemaphoreType.DMA,
        ],
    )
    def edges(hs_hbm, src_hbm, dst_hbm, out_hbm, sidx, didx0, didx1,
              rows0, rows1, acc, sem0, sem1, isem0, isem1):
        cid = lax.axis_index("c")
        sid = lax.axis_index("s")
        wid = cid * NS + sid
        wbase = wid * per

        pltpu.sync_copy(src_hbm.at[pl.ds(wbase, per)], sidx)

        def zb_body(r, carry):
            for k in range(D // LANES):
                rows0[r, pl.ds(k * LANES, LANES)] = jnp.zeros((LANES,), jnp.float32)
            return carry
        lax.fori_loop(0, CP, zb_body, 0)

        def zero_acc(kk, carry):
            pltpu.sync_copy(rows0, acc.at[pl.ds(sid * RPS + kk * CP, CP)])
            return carry
        lax.fori_loop(0, RPS // CP, zero_acc, 0)
        plsc.subcore_barrier()

        pltpu.async_copy(hs_hbm.at[sidx.at[pl.ds(0, C)]], rows0, sem0)
        pltpu.async_copy(dst_hbm.at[pl.ds(wbase, C)], didx0, isem0)

        def body(jj, carry):
            j0 = 2 * jj
            pltpu.async_copy(hs_hbm.at[sidx.at[pl.ds((j0 + 1) * C, C)]], rows1, sem1)
            pltpu.async_copy(dst_hbm.at[pl.ds(wbase + (j0 + 1) * C, C)], didx1, isem1)
            pltpu.make_async_copy(hs_hbm.at[sidx.at[pl.ds(j0 * C, C)]], rows0, sem0).wait()
            pltpu.make_async_copy(dst_hbm.at[pl.ds(wbase, C)], didx0, isem0).wait()
            pltpu.sync_copy(rows0, acc.at[didx0], add=True)
            pltpu.async_copy(hs_hbm.at[sidx.at[pl.ds((j0 + 2) * C, C)]], rows0, sem0)
            pltpu.async_copy(dst_hbm.at[pl.ds(wbase + (j0 + 2) * C, C)], didx0, isem0)
            pltpu.make_async_copy(hs_hbm.at[sidx.at[pl.ds((j0 + 1) * C, C)]], rows1, sem1).wait()
            pltpu.make_async_copy(dst_hbm.at[pl.ds(wbase, C)], didx1, isem1).wait()
            pltpu.sync_copy(rows1, acc.at[didx1], add=True)
            return carry
        lax.fori_loop(0, (NCH - 1) // 2, body, 0)

        pltpu.make_async_copy(hs_hbm.at[sidx.at[pl.ds((NCH - 1) * C, C)]], rows0, sem0).wait()
        pltpu.make_async_copy(dst_hbm.at[pl.ds(wbase, C)], didx0, isem0).wait()
        pltpu.sync_copy(rows0, acc.at[didx0], add=True)
        plsc.subcore_barrier()

        def copy_out(kk, carry):
            r0 = sid * RPS + kk * CP
            pltpu.sync_copy(acc.at[pl.ds(r0, CP)], rows0)
            pltpu.sync_copy(rows0, out_hbm.at[pl.ds(cid * NPAD + r0, CP)])
            return carry
        lax.fori_loop(0, RPS // CP, copy_out, 0)

    return edges


def _tc_pre(x, W, d0, d1, BM=2000):
    M, K = x.shape
    DO = W.shape[1]

    def body(x_ref, w_ref, d0_ref, d1_ref, hs_ref, dinv_ref):
        dinv = lax.rsqrt(d0_ref[...] + d1_ref[...] + 1.0)
        h = jnp.dot(x_ref[...], w_ref[...], preferred_element_type=jnp.float32)
        hs_ref[...] = h * dinv
        dinv_ref[...] = dinv

    return pl.pallas_call(
        body,
        grid=(M // BM,),
        in_specs=[
            pl.BlockSpec((BM, K), lambda i: (i, 0)),
            pl.BlockSpec((K, DO), lambda i: (0, 0)),
            pl.BlockSpec((BM, 1), lambda i: (i, 0)),
            pl.BlockSpec((BM, 1), lambda i: (i, 0)),
        ],
        out_specs=[
            pl.BlockSpec((BM, DO), lambda i: (i, 0)),
            pl.BlockSpec((BM, 1), lambda i: (i, 0)),
        ],
        out_shape=[
            jax.ShapeDtypeStruct((M, DO), jnp.float32),
            jax.ShapeDtypeStruct((M, 1), jnp.float32),
        ],
    )(x, W, d0, d1)


def _tc_mid(a0, a1, hsp, dinv, b, W, BM=2000):
    M, K = hsp.shape
    DO = W.shape[1]

    def body(a0_ref, a1_ref, hsp_ref, dinv_ref, b_ref, w_ref, hs_ref):
        x = (a0_ref[...] + a1_ref[...] + hsp_ref[...]) * dinv_ref[...] + b_ref[...]
        x = jnp.maximum(x, 0.0)
        h = jnp.dot(x, w_ref[...], preferred_element_type=jnp.float32)
        hs_ref[...] = h * dinv_ref[...]

    return pl.pallas_call(
        body,
        grid=(M // BM,),
        in_specs=[
            pl.BlockSpec((BM, K), lambda i: (i, 0)),
            pl.BlockSpec((BM, K), lambda i: (i, 0)),
            pl.BlockSpec((BM, K), lambda i: (i, 0)),
            pl.BlockSpec((BM, 1), lambda i: (i, 0)),
            pl.BlockSpec((1, K), lambda i: (0, 0)),
            pl.BlockSpec((K, DO), lambda i: (0, 0)),
        ],
        out_specs=pl.BlockSpec((BM, DO), lambda i: (i, 0)),
        out_shape=jax.ShapeDtypeStruct((M, DO), jnp.float32),
    )(a0, a1, hsp, dinv, b, W)


def _tc_final(a0, a1, hsp, dinv, b, BM=2000):
    M, K = hsp.shape

    def body(a0_ref, a1_ref, hsp_ref, dinv_ref, b_ref, out_ref):
        t = (a0_ref[...] + a1_ref[...] + hsp_ref[...]) * dinv_ref[...] + b_ref[...]
        m = jnp.max(t, axis=1, keepdims=True)
        lse = jnp.log(jnp.sum(jnp.exp(t - m), axis=1, keepdims=True)) + m
        out_ref[...] = t - lse

    return pl.pallas_call(
        body,
        grid=(M // BM,),
        in_specs=[
            pl.BlockSpec((BM, K), lambda i: (i, 0)),
            pl.BlockSpec((BM, K), lambda i: (i, 0)),
            pl.BlockSpec((BM, K), lambda i: (i, 0)),
            pl.BlockSpec((BM, 1), lambda i: (i, 0)),
            pl.BlockSpec((1, K), lambda i: (0, 0)),
        ],
        out_specs=pl.BlockSpec((BM, K), lambda i: (i, 0)),
        out_shape=jax.ShapeDtypeStruct((M, K), jnp.float32),
    )(a0, a1, hsp, dinv, b)


def kernel(features, adj, W1, b1, W2, b2, W3, b3):
    N, _ = features.shape
    E = adj.shape[1]
    per = E // NW
    C = _edge_chunk(per)
    NCH = per // C
    src = adj[0]
    dst = adj[1]

    NP2 = ((N + NS * 8 - 1) // (NS * 8)) * (NS * 8)
    NPAD = ((N + NS * 128 - 1) // (NS * 128)) * (NS * 128)

    deg2 = _deg_kernel(E, NP2)(dst)
    d0 = deg2[:N].reshape(N, 1)
    d1 = deg2[NP2:NP2 + N].reshape(N, 1)

    b1r = b1.reshape(1, -1)
    b2r = b2.reshape(1, -1)
    b3r = b3.reshape(1, -1)

    hs1, dinv = _tc_pre(features, W1, d0, d1)
    D = hs1.shape[1]
    edge_fn = _edge_kernel(N, D, E, NPAD)

    acc1 = edge_fn(hs1, src, dst)
    hs2 = _tc_mid(acc1[:N], acc1[NPAD:NPAD + N], hs1, dinv, b1r, W2)
    acc2 = edge_fn(hs2, src, dst)
    hs3 = _tc_mid(acc2[:N], acc2[NPAD:NPAD + N], hs2, dinv, b2r, W3)
    acc3 = edge_fn(hs3, src, dst)
    return _tc_final(acc3[:N], acc3[NPAD:NPAD + N], hs3, dinv, b3r)


# speedup vs baseline: 1.1156x; 1.1156x over previous
import functools

import jax
import jax.numpy as jnp
from jax import lax
from jax.experimental import pallas as pl
from jax.experimental.pallas import tpu as pltpu
from jax.experimental.pallas import tpu_sc as plsc

NC = 2
NS = 16
NW = NC * NS
LANES = 16


def _mesh():
    return plsc.VectorSubcoreMesh(
        core_axis_name="c", subcore_axis_name="s",
        num_cores=NC, num_subcores=NS)


def _edge_chunk(per):
    for c in range(128, 0, -8):
        if per % c == 0:
            return c
    raise ValueError(f"per-worker edge count {per} has no 8-multiple divisor")


def _deg_kernel(E, NP2):
    per = E // NW
    C = 128
    NF = per // C
    T = per - NF * C
    RPS = NP2 // NS

    @functools.partial(
        pl.kernel,
        out_type=jax.ShapeDtypeStruct((NC * NP2,), jnp.float32),
        mesh=_mesh(),
        scratch_types=[
            pltpu.VMEM((C,), jnp.int32),
            pltpu.VMEM((C,), jnp.int32),
            pltpu.VMEM((max(T, 8),), jnp.int32),
            pltpu.VMEM((C,), jnp.float32),
            pltpu.VMEM((max(T, LANES),), jnp.float32),
            pltpu.VMEM((RPS,), jnp.float32),
            pltpu.VMEM_SHARED((NP2,), jnp.float32),
            pltpu.SemaphoreType.DMA,
            pltpu.SemaphoreType.DMA,
        ],
    )
    def deg(dst_hbm, out_hbm, didx0, didx1, tidx, ones, tones, zb, acc,
            isem0, isem1):
        cid = lax.axis_index("c")
        sid = lax.axis_index("s")
        wid = cid * NS + sid
        wbase = wid * per
        fbase = wbase + T
        for k in range(C // LANES):
            ones[pl.ds(k * LANES, LANES)] = jnp.full((LANES,), 1.0, jnp.float32)

        def zb_body(k, carry):
            zb[pl.ds(k * LANES, LANES)] = jnp.zeros((LANES,), jnp.float32)
            return carry
        lax.fori_loop(0, RPS // LANES, zb_body, 0)
        pltpu.sync_copy(zb, acc.at[pl.ds(sid * RPS, RPS)])
        plsc.subcore_barrier()

        if T:
            for k in range(max(T, LANES) // LANES):
                tones[pl.ds(k * LANES, LANES)] = jnp.full((LANES,), 1.0, jnp.float32)
            pltpu.sync_copy(dst_hbm.at[pl.ds(wbase, T)], tidx)
            pltpu.sync_copy(tones, acc.at[tidx], add=True)
        pltpu.async_copy(dst_hbm.at[pl.ds(fbase, C)], didx0, isem0)

        def body(jj, carry):
            j0 = 2 * jj
            pltpu.async_copy(dst_hbm.at[pl.ds(fbase + (j0 + 1) * C, C)], didx1, isem1)
            pltpu.make_async_copy(dst_hbm.at[pl.ds(fbase, C)], didx0, isem0).wait()
            pltpu.sync_copy(ones, acc.at[didx0], add=True)

            @pl.when(j0 + 2 < NF)
            def _():
                pltpu.async_copy(dst_hbm.at[pl.ds(fbase + (j0 + 2) * C, C)], didx0, isem0)
            pltpu.make_async_copy(dst_hbm.at[pl.ds(fbase, C)], didx1, isem1).wait()
            pltpu.sync_copy(ones, acc.at[didx1], add=True)
            return carry
        assert NF % 2 == 0
        lax.fori_loop(0, NF // 2, body, 0)
        plsc.subcore_barrier()

        pltpu.sync_copy(acc.at[pl.ds(sid * RPS, RPS)], zb)
        pltpu.sync_copy(zb, out_hbm.at[pl.ds(cid * NP2 + sid * RPS, RPS)])

    return deg


def _edge_kernel(N, D, E, NPAD):
    per = E // NW
    C = 128
    NF = per // C
    T = per - NF * C
    RPS = NPAD // NS
    CP = C
    assert RPS % CP == 0 and NF % 2 == 0 and T % 8 == 0

    @functools.partial(
        pl.kernel,
        out_type=jax.ShapeDtypeStruct((NC * NPAD, D), jnp.float32),
        mesh=_mesh(),
        scratch_types=[
            pltpu.VMEM((per,), jnp.int32),
            pltpu.VMEM((C,), jnp.int32),
            pltpu.VMEM((C,), jnp.int32),
            pltpu.VMEM((max(T, 8),), jnp.int32),
            pltpu.VMEM((max(T, 8),), jnp.int32),
            pltpu.VMEM((max(T, 8), D), jnp.float32),
            pltpu.VMEM((C, D), jnp.float32),
            pltpu.VMEM((C, D), jnp.float32),
            pltpu.VMEM_SHARED((NPAD, D), jnp.float32),
            pltpu.SemaphoreType.DMA,
            pltpu.SemaphoreType.DMA,
            pltpu.SemaphoreType.DMA,
            pltpu.SemaphoreType.DMA,
        ],
    )
    def edges(hs_hbm, src_hbm, dst_hbm, out_hbm, sidx, didx0, didx1, tidx,
              tsidx, trow, rows0, rows1, acc, sem0, sem1, isem0, isem1):
        cid = lax.axis_index("c")
        sid = lax.axis_index("s")
        wid = cid * NS + sid
        wbase = wid * per
        fbase = wbase + T

        pltpu.async_copy(src_hbm.at[pl.ds(wbase, per)], sidx, isem0)

        def zb_body(r, carry):
            for k in range(D // LANES):
                rows0[r, pl.ds(k * LANES, LANES)] = jnp.zeros((LANES,), jnp.float32)
            return carry
        lax.fori_loop(0, CP, zb_body, 0)

        for kk in range(RPS // CP):
            pltpu.async_copy(rows0, acc.at[pl.ds(sid * RPS + kk * CP, CP)], sem0)
        pltpu.make_async_copy(src_hbm.at[pl.ds(wbase, per)], sidx, isem0).wait()
        for kk in range(RPS // CP):
            pltpu.make_async_copy(rows0, acc.at[pl.ds(sid * RPS, CP)], sem0).wait()
        plsc.subcore_barrier()

        if T:
            pltpu.sync_copy(dst_hbm.at[pl.ds(wbase, T)], tidx)
            pltpu.sync_copy(src_hbm.at[pl.ds(wbase, T)], tsidx)
            pltpu.async_copy(hs_hbm.at[tsidx], trow, sem0)
            pltpu.make_async_copy(hs_hbm.at[tsidx], trow, sem0).wait()
            pltpu.sync_copy(trow, acc.at[tidx], add=True)

        pltpu.async_copy(hs_hbm.at[sidx.at[pl.ds(T, C)]], rows0, sem0)
        pltpu.async_copy(dst_hbm.at[pl.ds(fbase, C)], didx0, isem0)

        def body(jj, carry):
            j0 = 2 * jj
            pltpu.async_copy(hs_hbm.at[sidx.at[pl.ds(T + (j0 + 1) * C, C)]], rows1, sem1)
            pltpu.async_copy(dst_hbm.at[pl.ds(fbase + (j0 + 1) * C, C)], didx1, isem1)
            pltpu.make_async_copy(hs_hbm.at[sidx.at[pl.ds(T, C)]], rows0, sem0).wait()
            pltpu.make_async_copy(dst_hbm.at[pl.ds(fbase, C)], didx0, isem0).wait()
            pltpu.sync_copy(rows0, acc.at[didx0], add=True)

            @pl.when(j0 + 2 < NF)
            def _():
                pltpu.async_copy(hs_hbm.at[sidx.at[pl.ds(T + (j0 + 2) * C, C)]], rows0, sem0)
                pltpu.async_copy(dst_hbm.at[pl.ds(fbase + (j0 + 2) * C, C)], didx0, isem0)
            pltpu.make_async_copy(hs_hbm.at[sidx.at[pl.ds(T, C)]], rows1, sem1).wait()
            pltpu.make_async_copy(dst_hbm.at[pl.ds(fbase, C)], didx1, isem1).wait()
            pltpu.sync_copy(rows1, acc.at[didx1], add=True)
            return carry
        lax.fori_loop(0, NF // 2, body, 0)
        plsc.subcore_barrier()

        nco = RPS // CP
        for kk in range(nco):
            buf = rows0 if kk % 2 == 0 else rows1
            sem = sem0 if kk % 2 == 0 else sem1
            if kk >= 2:
                pltpu.make_async_copy(buf, out_hbm.at[pl.ds(cid * NPAD, CP)], sem).wait()
            r0 = sid * RPS + kk * CP
            pltpu.sync_copy(acc.at[pl.ds(r0, CP)], buf)
            pltpu.async_copy(buf, out_hbm.at[pl.ds(cid * NPAD + r0, CP)], sem)
        for kk in (nco - 2, nco - 1):
            buf = rows0 if kk % 2 == 0 else rows1
            sem = sem0 if kk % 2 == 0 else sem1
            pltpu.make_async_copy(buf, out_hbm.at[pl.ds(cid * NPAD, CP)], sem).wait()

    return edges


def _tc_pre(x, W, d0, d1, BM=2000):
    M, K = x.shape
    DO = W.shape[1]

    def body(x_ref, w_ref, d0_ref, d1_ref, hs_ref, dinv_ref):
        dinv = lax.rsqrt(d0_ref[...] + d1_ref[...] + 1.0)
        h = jnp.dot(x_ref[...], w_ref[...], preferred_element_type=jnp.float32)
        hs_ref[...] = h * dinv
        dinv_ref[...] = dinv

    return pl.pallas_call(
        body,
        grid=(M // BM,),
        in_specs=[
            pl.BlockSpec((BM, K), lambda i: (i, 0)),
            pl.BlockSpec((K, DO), lambda i: (0, 0)),
            pl.BlockSpec((BM, 1), lambda i: (i, 0)),
            pl.BlockSpec((BM, 1), lambda i: (i, 0)),
        ],
        out_specs=[
            pl.BlockSpec((BM, DO), lambda i: (i, 0)),
            pl.BlockSpec((BM, 1), lambda i: (i, 0)),
        ],
        out_shape=[
            jax.ShapeDtypeStruct((M, DO), jnp.float32),
            jax.ShapeDtypeStruct((M, 1), jnp.float32),
        ],
    )(x, W, d0, d1)


def _tc_mid(a0, a1, hsp, dinv, b, W, BM=2000):
    M, K = hsp.shape
    DO = W.shape[1]

    def body(a0_ref, a1_ref, hsp_ref, dinv_ref, b_ref, w_ref, hs_ref):
        x = (a0_ref[...] + a1_ref[...] + hsp_ref[...]) * dinv_ref[...] + b_ref[...]
        x = jnp.maximum(x, 0.0)
        h = jnp.dot(x, w_ref[...], preferred_element_type=jnp.float32)
        hs_ref[...] = h * dinv_ref[...]

    return pl.pallas_call(
        body,
        grid=(M // BM,),
        in_specs=[
            pl.BlockSpec((BM, K), lambda i: (i, 0)),
            pl.BlockSpec((BM, K), lambda i: (i, 0)),
            pl.BlockSpec((BM, K), lambda i: (i, 0)),
            pl.BlockSpec((BM, 1), lambda i: (i, 0)),
            pl.BlockSpec((1, K), lambda i: (0, 0)),
            pl.BlockSpec((K, DO), lambda i: (0, 0)),
        ],
        out_specs=pl.BlockSpec((BM, DO), lambda i: (i, 0)),
        out_shape=jax.ShapeDtypeStruct((M, DO), jnp.float32),
    )(a0, a1, hsp, dinv, b, W)


def _tc_final(a0, a1, hsp, dinv, b, BM=2000):
    M, K = hsp.shape

    def body(a0_ref, a1_ref, hsp_ref, dinv_ref, b_ref, out_ref):
        t = (a0_ref[...] + a1_ref[...] + hsp_ref[...]) * dinv_ref[...] + b_ref[...]
        m = jnp.max(t, axis=1, keepdims=True)
        lse = jnp.log(jnp.sum(jnp.exp(t - m), axis=1, keepdims=True)) + m
        out_ref[...] = t - lse

    return pl.pallas_call(
        body,
        grid=(M // BM,),
        in_specs=[
            pl.BlockSpec((BM, K), lambda i: (i, 0)),
            pl.BlockSpec((BM, K), lambda i: (i, 0)),
            pl.BlockSpec((BM, K), lambda i: (i, 0)),
            pl.BlockSpec((BM, 1), lambda i: (i, 0)),
            pl.BlockSpec((1, K), lambda i: (0, 0)),
        ],
        out_specs=pl.BlockSpec((BM, K), lambda i: (i, 0)),
        out_shape=jax.ShapeDtypeStruct((M, K), jnp.float32),
    )(a0, a1, hsp, dinv, b)


def kernel(features, adj, W1, b1, W2, b2, W3, b3):
    N, _ = features.shape
    E = adj.shape[1]
    per = E // NW
    C = _edge_chunk(per)
    NCH = per // C
    src = adj[0]
    dst = adj[1]

    NP2 = ((N + NS * 8 - 1) // (NS * 8)) * (NS * 8)
    NPAD = ((N + NS * 128 - 1) // (NS * 128)) * (NS * 128)

    deg2 = _deg_kernel(E, NP2)(dst)
    d0 = deg2[:N].reshape(N, 1)
    d1 = deg2[NP2:NP2 + N].reshape(N, 1)

    b1r = b1.reshape(1, -1)
    b2r = b2.reshape(1, -1)
    b3r = b3.reshape(1, -1)

    hs1, dinv = _tc_pre(features, W1, d0, d1)
    D = hs1.shape[1]
    edge_fn = _edge_kernel(N, D, E, NPAD)

    acc1 = edge_fn(hs1, src, dst)
    hs2 = _tc_mid(acc1[:N], acc1[NPAD:NPAD + N], hs1, dinv, b1r, W2)
    acc2 = edge_fn(hs2, src, dst)
    hs3 = _tc_mid(acc2[:N], acc2[NPAD:NPAD + N], hs2, dinv, b2r, W3)
    acc3 = edge_fn(hs3, src, dst)
    return _tc_final(acc3[:N], acc3[NPAD:NPAD + N], hs3, dinv, b3r)
